# baseline (speedup 1.0000x reference)
import functools

import jax
import jax.numpy as jnp
from jax import lax
from jax.experimental import pallas as pl
from jax.experimental.pallas import tpu as pltpu

_EPS = 1e-5
_INST_PER_GROUP = 16
_OUT_SPLITS = 4


def _norm_body(x_ref, o_ref, *, d, inv_s, n_out):
    j = pl.program_id(1)
    base = j * (n_out * d)
    for k in range(n_out):
        x = x_ref[pl.ds(base + k * d, d)]
        p0 = jnp.sum(x, axis=0)
        q0 = jnp.sum(x * x, axis=0)
        p1 = jnp.sum(p0, axis=0, keepdims=True)
        q1 = jnp.sum(q0, axis=0, keepdims=True)
        s = jnp.sum(p1, axis=-1, keepdims=True)
        q = jnp.sum(q1, axis=-1, keepdims=True)
        mean = s * inv_s
        var = jnp.maximum(q * inv_s - mean * mean, 0.0)
        scale = lax.rsqrt(var + _EPS)
        shift = -mean * scale
        o_ref[k * d:(k + 1) * d] = x * scale + shift


def _instance_norm(x3, r, d, h, w, n_grp, n_split):
    n_out = n_grp // n_split
    grp_rows = d * n_grp
    out_rows = d * n_out
    return pl.pallas_call(
        functools.partial(_norm_body, d=d, inv_s=1.0 / (d * h * w),
                          n_out=n_out),
        out_shape=jax.ShapeDtypeStruct(x3.shape, x3.dtype),
        grid=(r // n_grp, n_split),
        in_specs=[pl.BlockSpec((grp_rows, h, w), lambda i, j: (i, 0, 0))],
        out_specs=pl.BlockSpec((out_rows, h, w),
                               lambda i, j, _s=n_split: (i * _s + j, 0, 0)),
        compiler_params=pltpu.CompilerParams(
            dimension_semantics=("parallel", "arbitrary"),
        ),
    )(x3)


def kernel(x):
    n, c, d, h, w = x.shape
    r = n * c
    n_grp = _INST_PER_GROUP
    while r % n_grp:
        n_grp //= 2
    n_split = _OUT_SPLITS
    while n_grp % n_split:
        n_split //= 2
    x3 = x.reshape(r * d, h, w)
    out = _instance_norm(x3, r, d, h, w, n_grp, n_split)
    return out.reshape(n, c, d, h, w)
